# speedup vs baseline: 3.0467x; 3.0467x over previous
import functools

import jax
import jax.numpy as jnp
from jax import lax
from jax.experimental import pallas as pl
from jax.experimental.pallas import tpu as pltpu
from jax.experimental.pallas import tpu_sc as plsc

_D = 128
_CHUNK = 128
_NW = 32


def _make_gather(n_rows):
    b_per_w = n_rows // _NW
    n_chunks = b_per_w // _CHUNK
    mesh = plsc.VectorSubcoreMesh(core_axis_name="c", subcore_axis_name="s")

    @functools.partial(
        pl.kernel,
        mesh=mesh,
        out_type=jax.ShapeDtypeStruct((n_rows, _D), jnp.float32),
        scratch_types=[
            pltpu.VMEM((n_chunks, _CHUNK), jnp.int32),
            pltpu.VMEM((_CHUNK, _D), jnp.float32),
            pltpu.SemaphoreType.DMA,
        ],
    )
    def gather_kernel(table_hbm, idx_hbm, out_hbm, idx_v, rows_v, sem):
        cid = lax.axis_index("c")
        sid = lax.axis_index("s")
        wid = sid * 2 + cid
        base = wid * b_per_w
        pltpu.sync_copy(idx_hbm.at[pl.ds(wid * n_chunks, n_chunks)], idx_v)

        def body(j, carry):
            pltpu.async_copy(table_hbm.at[idx_v.at[j]], rows_v, sem).wait()
            pltpu.sync_copy(rows_v, out_hbm.at[pl.ds(base + j * _CHUNK, _CHUNK)])
            return carry

        lax.fori_loop(0, n_chunks, body, 0)

    return gather_kernel


def kernel(input_ids, embed_table):
    batch, hist = input_ids.shape
    flat = input_ids.reshape(-1).astype(jnp.int32)
    idx2d = flat.reshape(-1, _CHUNK)
    out = _make_gather(batch * hist)(embed_table, idx2d)
    return out.reshape(batch, hist, _D)


# speedup vs baseline: 3.4554x; 1.1342x over previous
import functools

import jax
import jax.numpy as jnp
from jax import lax
from jax.experimental import pallas as pl
from jax.experimental.pallas import tpu as pltpu
from jax.experimental.pallas import tpu_sc as plsc

_D = 128
_CHUNK = 128
_NW = 32
_NBUF = 4
_DEPTH = 2


def _make_gather(n_rows):
    b_per_w = n_rows // _NW
    n_chunks = b_per_w // _CHUNK
    mesh = plsc.VectorSubcoreMesh(core_axis_name="c", subcore_axis_name="s")

    @functools.partial(
        pl.kernel,
        mesh=mesh,
        out_type=jax.ShapeDtypeStruct((n_rows, _D), jnp.float32),
        scratch_types=[
            pltpu.VMEM((n_chunks, _CHUNK), jnp.int32),
            pltpu.VMEM((_NBUF, _CHUNK, _D), jnp.float32),
            pltpu.SemaphoreType.DMA((_NBUF,)),
            pltpu.SemaphoreType.DMA((_NBUF,)),
        ],
    )
    def gather_kernel(table_hbm, idx_hbm, out_hbm, idx_v, rows_v, gsem, psem):
        cid = lax.axis_index("c")
        sid = lax.axis_index("s")
        wid = sid * 2 + cid
        base = wid * b_per_w
        pltpu.sync_copy(idx_hbm.at[pl.ds(wid * n_chunks, n_chunks)], idx_v)

        for b in range(_DEPTH):
            pltpu.async_copy(table_hbm.at[idx_v.at[b]], rows_v.at[b], gsem.at[b])

        def body(j, carry):
            b = lax.rem(j, _NBUF)
            pltpu.make_async_copy(
                table_hbm.at[idx_v.at[j]], rows_v.at[b], gsem.at[b]
            ).wait()
            pltpu.async_copy(
                rows_v.at[b],
                out_hbm.at[pl.ds(base + j * _CHUNK, _CHUNK)],
                psem.at[b],
            )
            jn = j + _DEPTH

            @pl.when(jn < n_chunks)
            def _():
                bn = lax.rem(jn, _NBUF)

                @pl.when(jn >= _NBUF)
                def _():
                    jo = jn - _NBUF
                    pltpu.make_async_copy(
                        rows_v.at[bn],
                        out_hbm.at[pl.ds(base + jo * _CHUNK, _CHUNK)],
                        psem.at[bn],
                    ).wait()

                pltpu.async_copy(table_hbm.at[idx_v.at[jn]], rows_v.at[bn], gsem.at[bn])

            return carry

        lax.fori_loop(0, n_chunks, body, 0)

        for t in range(_NBUF):
            jo = n_chunks - _NBUF + t
            b = jo % _NBUF
            pltpu.make_async_copy(
                rows_v.at[b],
                out_hbm.at[pl.ds(base + jo * _CHUNK, _CHUNK)],
                psem.at[b],
            ).wait()

    return gather_kernel


def kernel(input_ids, embed_table):
    batch, hist = input_ids.shape
    flat = input_ids.reshape(-1).astype(jnp.int32)
    idx2d = flat.reshape(-1, _CHUNK)
    out = _make_gather(batch * hist)(embed_table, idx2d)
    return out.reshape(batch, hist, _D)


# speedup vs baseline: 3.4605x; 1.0015x over previous
import functools

import jax
import jax.numpy as jnp
from jax import lax
from jax.experimental import pallas as pl
from jax.experimental.pallas import tpu as pltpu
from jax.experimental.pallas import tpu_sc as plsc

_D = 128
_CHUNK = 128
_NW = 32
_NBUF = 6
_DEPTH = 4


def _make_gather(n_rows):
    b_per_w = n_rows // _NW
    n_chunks = b_per_w // _CHUNK
    mesh = plsc.VectorSubcoreMesh(core_axis_name="c", subcore_axis_name="s")

    @functools.partial(
        pl.kernel,
        mesh=mesh,
        out_type=jax.ShapeDtypeStruct((n_rows, _D), jnp.float32),
        scratch_types=[
            pltpu.VMEM((n_chunks, _CHUNK), jnp.int32),
            pltpu.VMEM((_NBUF, _CHUNK, _D), jnp.float32),
            pltpu.SemaphoreType.DMA((_NBUF,)),
            pltpu.SemaphoreType.DMA((_NBUF,)),
        ],
    )
    def gather_kernel(table_hbm, idx_hbm, out_hbm, idx_v, rows_v, gsem, psem):
        cid = lax.axis_index("c")
        sid = lax.axis_index("s")
        wid = sid * 2 + cid
        base = wid * b_per_w
        pltpu.sync_copy(idx_hbm.at[pl.ds(wid * n_chunks, n_chunks)], idx_v)

        for b in range(_DEPTH):
            pltpu.async_copy(table_hbm.at[idx_v.at[b]], rows_v.at[b], gsem.at[b])

        def body(j, carry):
            b = lax.rem(j, _NBUF)
            pltpu.make_async_copy(
                table_hbm.at[idx_v.at[j]], rows_v.at[b], gsem.at[b]
            ).wait()
            pltpu.async_copy(
                rows_v.at[b],
                out_hbm.at[pl.ds(base + j * _CHUNK, _CHUNK)],
                psem.at[b],
            )
            jn = j + _DEPTH

            @pl.when(jn < n_chunks)
            def _():
                bn = lax.rem(jn, _NBUF)

                @pl.when(jn >= _NBUF)
                def _():
                    jo = jn - _NBUF
                    pltpu.make_async_copy(
                        rows_v.at[bn],
                        out_hbm.at[pl.ds(base + jo * _CHUNK, _CHUNK)],
                        psem.at[bn],
                    ).wait()

                pltpu.async_copy(table_hbm.at[idx_v.at[jn]], rows_v.at[bn], gsem.at[bn])

            return carry

        lax.fori_loop(0, n_chunks, body, 0)

        for t in range(_NBUF):
            jo = n_chunks - _NBUF + t
            b = jo % _NBUF
            pltpu.make_async_copy(
                rows_v.at[b],
                out_hbm.at[pl.ds(base + jo * _CHUNK, _CHUNK)],
                psem.at[b],
            ).wait()

    return gather_kernel


def kernel(input_ids, embed_table):
    batch, hist = input_ids.shape
    flat = input_ids.reshape(-1).astype(jnp.int32)
    idx2d = flat.reshape(-1, _CHUNK)
    out = _make_gather(batch * hist)(embed_table, idx2d)
    return out.reshape(batch, hist, _D)


# speedup vs baseline: 6.4009x; 1.8497x over previous
import functools

import jax
import jax.numpy as jnp
from jax import lax
from jax.experimental import pallas as pl
from jax.experimental.pallas import tpu as pltpu
from jax.experimental.pallas import tpu_sc as plsc

_D = 128
_NW = 32
_GRP = 2
_NBUF = 4
_DEPTH = 2


def _make_gather(batch, hist):
    rows_per_w = batch // _NW
    n_chunks = rows_per_w // _GRP
    mesh = plsc.VectorSubcoreMesh(core_axis_name="c", subcore_axis_name="s")

    @functools.partial(
        pl.kernel,
        mesh=mesh,
        out_type=jax.ShapeDtypeStruct((batch, hist, _D), jnp.float32),
        scratch_types=[
            pltpu.VMEM((rows_per_w, hist), jnp.int32),
            pltpu.VMEM((_NBUF, _GRP, hist, _D), jnp.float32),
            pltpu.SemaphoreType.DMA((_NBUF,)),
            pltpu.SemaphoreType.DMA((_NBUF,)),
        ],
    )
    def gather_kernel(table_hbm, idx_hbm, out_hbm, idx_v, rows_v, gsem, psem):
        cid = lax.axis_index("c")
        sid = lax.axis_index("s")
        wid = sid * 2 + cid
        base = wid * rows_per_w
        pltpu.sync_copy(idx_hbm.at[pl.ds(base, rows_per_w)], idx_v)

        def fire_gathers(j, b):
            for r in range(_GRP):
                pltpu.async_copy(
                    table_hbm.at[idx_v.at[j * _GRP + r]],
                    rows_v.at[b].at[r],
                    gsem.at[b],
                )

        def wait_gathers(j, b):
            pltpu.make_async_copy(
                out_hbm.at[pl.ds(base + j * _GRP, _GRP)],
                rows_v.at[b],
                gsem.at[b],
            ).wait()

        def put_desc(j, b):
            return pltpu.make_async_copy(
                rows_v.at[b],
                out_hbm.at[pl.ds(base + j * _GRP, _GRP)],
                psem.at[b],
            )

        for b in range(_DEPTH):
            fire_gathers(b, b)

        def body(j, carry):
            b = lax.rem(j, _NBUF)
            wait_gathers(j, b)
            put_desc(j, b).start()
            jn = j + _DEPTH

            @pl.when(jn < n_chunks)
            def _():
                bn = lax.rem(jn, _NBUF)

                @pl.when(jn >= _NBUF)
                def _():
                    put_desc(jn - _NBUF, bn).wait()

                fire_gathers(jn, bn)

            return carry

        lax.fori_loop(0, n_chunks, body, 0)

        for t in range(_NBUF):
            jo = n_chunks - _NBUF + t
            put_desc(jo, jo % _NBUF).wait()

    return gather_kernel


def kernel(input_ids, embed_table):
    batch, hist = input_ids.shape
    idx = input_ids.astype(jnp.int32)
    return _make_gather(batch, hist)(embed_table, idx)
